# baseline (speedup 1.0000x reference)
import jax
import jax.numpy as jnp
from jax import lax
from jax.experimental import pallas as pl
from jax.experimental.pallas import tpu as pltpu
from jax.experimental.pallas import tpu_sc as plsc

DIM = 1024
E = 8
TOPK = 2
INTER = 512
SH = 1024
T = 2048
A = T * TOPK
BLK = 128
NB = A // BLK + E
P = NB * BLK
BT = 256
NC, NS = 2, 16
NW = NC * NS
TPW = T // NW
SPW = P // NW
GCH = SPW // 4
CCH = 32
SLOTC = 512
HD = DIM // 2


def _silu(v):
    return v * jax.nn.sigmoid(v)


def _mmT(a, b):
    return jax.lax.dot_general(a, b, (((1,), (1,)), ((), ())),
                               preferred_element_type=jnp.float32)


def _pack(v):
    u = lax.bitcast_convert_type(v, jnp.uint32)
    one = jnp.uint32(1)
    r = (u + jnp.uint32(0x7FFF) + ((u >> jnp.uint32(16)) & one)) >> jnp.uint32(16)
    w = r[:, :HD] | (r[:, HD:] << jnp.uint32(16))
    return lax.bitcast_convert_type(w, jnp.float32)


def _unpack(w):
    u = lax.bitcast_convert_type(w, jnp.uint32)
    lo = lax.bitcast_convert_type(u << jnp.uint32(16), jnp.float32)
    hi = lax.bitcast_convert_type(u & jnp.uint32(0xFFFF0000), jnp.float32)
    return jnp.concatenate([lo, hi], axis=1).astype(jnp.bfloat16)


def _cast_body(w1_ref, w3_ref, w2_ref, s1_ref, s3_ref, s2_ref,
               o1_ref, o3_ref, o2_ref, t1_ref, t3_ref, t2_ref):
    bf16 = jnp.bfloat16
    o1_ref[...] = w1_ref[...].astype(bf16)
    o3_ref[...] = w3_ref[...].astype(bf16)
    o2_ref[...] = w2_ref[...].astype(bf16)
    t1_ref[...] = s1_ref[...].astype(bf16)
    t3_ref[...] = s3_ref[...].astype(bf16)
    t2_ref[...] = s2_ref[...].astype(bf16)


def _route_body(x_ref, wg_ref, bias_ref,
                d0_ref, d1_ref, st_ref, sw_ref, be_ref, xw_ref):
    x = x_ref[...]
    xw_ref[...] = _pack(x)
    scores = jax.nn.sigmoid(_mmT(x, wg_ref[...]))
    biased = scores + bias_ref[...]
    lane = jax.lax.broadcasted_iota(jnp.int32, (T, E), 1)
    m0 = jnp.max(biased, axis=1, keepdims=True)
    i0 = jnp.min(jnp.where(biased == m0, lane, E), axis=1, keepdims=True)
    masked = jnp.where(lane == i0, -jnp.inf, biased)
    m1 = jnp.max(masked, axis=1, keepdims=True)
    i1 = jnp.min(jnp.where(masked == m1, lane, E), axis=1, keepdims=True)
    w0col = jnp.sum(jnp.where(lane == i0, scores, 0.0), axis=1,
                    keepdims=True)
    w1col = jnp.sum(jnp.where(lane == i1, scores, 0.0), axis=1,
                    keepdims=True)
    cnt = ((lane == i0).astype(jnp.float32)
           + (lane == i1).astype(jnp.float32))
    r2 = jax.lax.broadcasted_iota(jnp.int32, (T, T), 0)
    c2 = jax.lax.broadcasted_iota(jnp.int32, (T, T), 1)
    tri = (c2 <= r2).astype(jnp.bfloat16)
    inc = jax.lax.dot_general(tri, cnt.astype(jnp.bfloat16),
                              (((1,), (0,)), ((), ())),
                              preferred_element_type=jnp.float32)
    exc = inc - cnt
    counts = inc[T - 1:T, :]
    nb = jnp.floor((counts + (BLK - 1)) * (1.0 / BLK))
    r8 = jax.lax.broadcasted_iota(jnp.int32, (E, E), 0)
    c8 = jax.lax.broadcasted_iota(jnp.int32, (E, E), 1)
    su = (r8 < c8).astype(jnp.float32)
    offb = jax.lax.dot_general(nb, su, (((1,), (0,)), ((), ())),
                               preferred_element_type=jnp.float32)
    offs = offb * float(BLK)
    d0 = jnp.sum(jnp.where(lane == i0, exc + offs, 0.0), axis=1, keepdims=True)
    d1 = jnp.sum(jnp.where(lane == i1, exc + offs, 0.0), axis=1, keepdims=True)
    d0_ref[...] = d0.astype(jnp.int32)
    d1_ref[...] = d1.astype(jnp.int32)
    ti = jax.lax.broadcasted_iota(jnp.int32, (T, SLOTC), 0).astype(jnp.float32)
    qi = jax.lax.broadcasted_iota(jnp.int32, (T, SLOTC), 1).astype(jnp.float32)
    for cidx in range(P // SLOTC):
        q = qi + float(cidx * SLOTC)
        m0 = d0 == q
        m1 = d1 == q
        stv = jnp.sum(jnp.where(m0 | m1, ti, 0.0), axis=0, keepdims=True)
        swv = (jnp.sum(jnp.where(m0, w0col, 0.0), axis=0, keepdims=True)
               + jnp.sum(jnp.where(m1, w1col, 0.0), axis=0, keepdims=True))
        st_ref[pl.ds(cidx, 1), :] = stv.astype(jnp.int32)
        sw_ref[pl.ds(cidx, 1), :] = swv
    offb_col = jnp.sum(jnp.where(r8 == c8, jnp.broadcast_to(offb, (E, E)),
                                 0.0), axis=1, keepdims=True)
    jb = jax.lax.broadcasted_iota(jnp.int32, (E, NB), 1).astype(jnp.float32)
    be = jnp.sum((jb >= offb_col).astype(jnp.float32), axis=0,
                 keepdims=True) - 1.0
    be_ref[...] = be.astype(jnp.int32)


def _dispatch_body(stok_hbm, x_hbm, xs_hbm,
                   idx_v, rows0_v, rows1_v, sg0, sg1, sw0, sw1):
    c = lax.axis_index("c")
    s = lax.axis_index("s")
    slot0 = c * (P // NC) + s * SPW
    pltpu.sync_copy(stok_hbm.at[pl.ds(slot0, SPW)], idx_v)
    bufs = (rows0_v, rows1_v)
    gsems = (sg0, sg1)
    wsems = (sw0, sw1)
    nch = SPW // GCH
    gps = [None] * nch
    wrs = [None] * nch
    for k in range(nch):
        if k >= 2:
            wrs[k - 2].wait()
        gps[k] = pltpu.async_copy(x_hbm.at[idx_v.at[pl.ds(k * GCH, GCH)]],
                                  bufs[k % 2], gsems[k % 2])
        if k >= 1:
            gps[k - 1].wait()
            wrs[k - 1] = pltpu.async_copy(
                bufs[(k - 1) % 2],
                xs_hbm.at[pl.ds(slot0 + (k - 1) * GCH, GCH)],
                wsems[(k - 1) % 2])
    gps[nch - 1].wait()
    wrs[nch - 1] = pltpu.async_copy(
        bufs[(nch - 1) % 2], xs_hbm.at[pl.ds(slot0 + (nch - 1) * GCH, GCH)],
        wsems[(nch - 1) % 2])
    wrs[nch - 2].wait()
    wrs[nch - 1].wait()


def _sharedh_body(xw_ref, ws1_ref, ws3_ref, hs_ref):
    x = _unpack(xw_ref[...])
    h = _silu(_mmT(x, ws1_ref[...])) * _mmT(x, ws3_ref[...])
    hs_ref[...] = h.astype(jnp.bfloat16)


def _grouped_body(be_ref, xs_ref, w1_ref, w3_ref, w2_ref, wsl_ref, eo_ref):
    x = _unpack(xs_ref[...])
    h = _silu(_mmT(x, w1_ref[0])) * _mmT(x, w3_ref[0])
    eo = _mmT(h.astype(jnp.bfloat16), w2_ref[0])
    eo_ref[...] = _pack(eo * wsl_ref[0])


def _combine_body(d0_hbm, d1_hbm, eos_hbm, y0_hbm, y1_hbm,
                  d0_v, d1_v, rows0_v, rows1_v, sg0, sg1, sw0, sw1):
    wid = lax.axis_index("c") * NS + lax.axis_index("s")
    base = wid * TPW
    pltpu.sync_copy(d0_hbm.at[pl.ds(base, TPW)], d0_v)
    pltpu.sync_copy(d1_hbm.at[pl.ds(base, TPW)], d1_v)
    nch = TPW // CCH
    units = ([(d0_v, y0_hbm, k) for k in range(nch)]
             + [(d1_v, y1_hbm, k) for k in range(nch)])
    bufs = (rows0_v, rows1_v)
    gsems = (sg0, sg1)
    wsems = (sw0, sw1)
    n = len(units)
    gps = [None] * n
    wrs = [None] * n
    for u in range(n):
        idx_v, out_hbm, k = units[u]
        if u >= 2:
            wrs[u - 2].wait()
        gps[u] = pltpu.async_copy(eos_hbm.at[idx_v.at[pl.ds(k * CCH, CCH)]],
                                  bufs[u % 2], gsems[u % 2])
        if u >= 1:
            pidx, pout, pk = units[u - 1]
            gps[u - 1].wait()
            wrs[u - 1] = pltpu.async_copy(
                bufs[(u - 1) % 2], pout.at[pl.ds(base + pk * CCH, CCH)],
                wsems[(u - 1) % 2])
    lidx, lout, lk = units[n - 1]
    gps[n - 1].wait()
    wrs[n - 1] = pltpu.async_copy(
        bufs[(n - 1) % 2], lout.at[pl.ds(base + lk * CCH, CCH)],
        wsems[(n - 1) % 2])
    wrs[n - 2].wait()
    wrs[n - 1].wait()


def _final_body(hs_ref, y0_ref, y1_ref, ws2_ref, o_ref):
    z = _mmT(hs_ref[...], ws2_ref[...])
    y0 = _unpack(y0_ref[...]).astype(jnp.float32)
    y1 = _unpack(y1_ref[...]).astype(jnp.float32)
    o_ref[...] = z + y0 + y1


@jax.jit
def _run(x, Wg, expert_bias, W1, W2, W3, Ws1, Ws2, Ws3):
    shape = x.shape
    xf = x.reshape(-1, DIM)
    bias2 = expert_bias.reshape(1, E)
    f32 = jnp.float32
    i32 = jnp.int32
    bf16 = jnp.bfloat16

    W1b, W3b, W2b, Ws1b, Ws3b, Ws2b = pl.pallas_call(
        _cast_body,
        grid=(E,),
        in_specs=[
            pl.BlockSpec((1, INTER, DIM), lambda i: (i, 0, 0)),
            pl.BlockSpec((1, INTER, DIM), lambda i: (i, 0, 0)),
            pl.BlockSpec((1, DIM, INTER), lambda i: (i, 0, 0)),
            pl.BlockSpec((SH // E, DIM), lambda i: (i, 0)),
            pl.BlockSpec((SH // E, DIM), lambda i: (i, 0)),
            pl.BlockSpec((DIM // E, SH), lambda i: (i, 0)),
        ],
        out_specs=[
            pl.BlockSpec((1, INTER, DIM), lambda i: (i, 0, 0)),
            pl.BlockSpec((1, INTER, DIM), lambda i: (i, 0, 0)),
            pl.BlockSpec((1, DIM, INTER), lambda i: (i, 0, 0)),
            pl.BlockSpec((SH // E, DIM), lambda i: (i, 0)),
            pl.BlockSpec((SH // E, DIM), lambda i: (i, 0)),
            pl.BlockSpec((DIM // E, SH), lambda i: (i, 0)),
        ],
        out_shape=(
            jax.ShapeDtypeStruct((E, INTER, DIM), bf16),
            jax.ShapeDtypeStruct((E, INTER, DIM), bf16),
            jax.ShapeDtypeStruct((E, DIM, INTER), bf16),
            jax.ShapeDtypeStruct((SH, DIM), bf16),
            jax.ShapeDtypeStruct((SH, DIM), bf16),
            jax.ShapeDtypeStruct((DIM, SH), bf16),
        ),
    )(W1, W3, W2, Ws1, Ws3, Ws2)

    d0, d1, st, sw, be, xw = pl.pallas_call(
        _route_body,
        out_shape=(
            jax.ShapeDtypeStruct((T, 1), i32),
            jax.ShapeDtypeStruct((T, 1), i32),
            jax.ShapeDtypeStruct((P // SLOTC, SLOTC), i32),
            jax.ShapeDtypeStruct((P // SLOTC, SLOTC), f32),
            jax.ShapeDtypeStruct((1, NB), i32),
            jax.ShapeDtypeStruct((T, HD), f32),
        ),
    )(xf, Wg, bias2)
    d0 = d0.reshape(T)
    d1 = d1.reshape(T)

    mesh = plsc.VectorSubcoreMesh(core_axis_name="c", subcore_axis_name="s",
                                  num_cores=NC, num_subcores=NS)
    xs = pl.kernel(
        _dispatch_body,
        out_type=jax.ShapeDtypeStruct((P, HD), f32),
        mesh=mesh,
        scratch_types=[
            pltpu.VMEM((SPW,), i32),
            pltpu.VMEM((GCH, HD), f32),
            pltpu.VMEM((GCH, HD), f32),
            pltpu.SemaphoreType.DMA,
            pltpu.SemaphoreType.DMA,
            pltpu.SemaphoreType.DMA,
            pltpu.SemaphoreType.DMA,
        ],
    )(st.reshape(P), xw)

    hs = pl.pallas_call(
        _sharedh_body,
        grid=(T // BT,),
        in_specs=[
            pl.BlockSpec((BT, HD), lambda i: (i, 0)),
            pl.BlockSpec((SH, DIM), lambda i: (0, 0)),
            pl.BlockSpec((SH, DIM), lambda i: (0, 0)),
        ],
        out_specs=pl.BlockSpec((BT, SH), lambda i: (i, 0)),
        out_shape=jax.ShapeDtypeStruct((T, SH), bf16),
    )(xw, Ws1b, Ws3b)

    eos = pl.pallas_call(
        _grouped_body,
        grid_spec=pltpu.PrefetchScalarGridSpec(
            num_scalar_prefetch=1,
            grid=(NB,),
            in_specs=[
                pl.BlockSpec((BLK, HD), lambda i, be: (i, 0)),
                pl.BlockSpec((1, INTER, DIM), lambda i, be: (be[i], 0, 0)),
                pl.BlockSpec((1, INTER, DIM), lambda i, be: (be[i], 0, 0)),
                pl.BlockSpec((1, DIM, INTER), lambda i, be: (be[i], 0, 0)),
                pl.BlockSpec((1, BLK, 1), lambda i, be: (i, 0, 0)),
            ],
            out_specs=pl.BlockSpec((BLK, HD), lambda i, be: (i, 0)),
        ),
        out_shape=jax.ShapeDtypeStruct((P, HD), f32),
    )(be.reshape(NB), xs, W1b, W3b, W2b, sw.reshape(NB, BLK, 1))

    y0, y1 = pl.kernel(
        _combine_body,
        out_type=(
            jax.ShapeDtypeStruct((T, HD), f32),
            jax.ShapeDtypeStruct((T, HD), f32),
        ),
        mesh=mesh,
        scratch_types=[
            pltpu.VMEM((TPW,), i32),
            pltpu.VMEM((TPW,), i32),
            pltpu.VMEM((CCH, HD), f32),
            pltpu.VMEM((CCH, HD), f32),
            pltpu.SemaphoreType.DMA,
            pltpu.SemaphoreType.DMA,
            pltpu.SemaphoreType.DMA,
            pltpu.SemaphoreType.DMA,
        ],
    )(d0, d1, eos)

    out = pl.pallas_call(
        _final_body,
        grid=(T // BT,),
        in_specs=[
            pl.BlockSpec((BT, SH), lambda i: (i, 0)),
            pl.BlockSpec((BT, HD), lambda i: (i, 0)),
            pl.BlockSpec((BT, HD), lambda i: (i, 0)),
            pl.BlockSpec((DIM, SH), lambda i: (0, 0)),
        ],
        out_specs=pl.BlockSpec((BT, DIM), lambda i: (i, 0)),
        out_shape=jax.ShapeDtypeStruct((T, DIM), f32),
    )(hs, y0, y1, Ws2b)

    return out.reshape(shape)


def kernel(x, Wg, expert_bias, W1, W2, W3, Ws1, Ws2, Ws3):
    return _run(x, Wg, expert_bias, W1, W2, W3, Ws1, Ws2, Ws3)


# speedup vs baseline: 1.4990x; 1.4990x over previous
import jax
import jax.numpy as jnp
from jax import lax
from jax.experimental import pallas as pl
from jax.experimental.pallas import tpu as pltpu
from jax.experimental.pallas import tpu_sc as plsc

DIM = 1024
E = 8
TOPK = 2
INTER = 512
SH = 1024
T = 2048
A = T * TOPK
BLK = 256
NB = A // BLK + E
P = NB * BLK
BT = 256
NC, NS = 2, 16
NW = NC * NS
TPW = T // NW
SPW = P // NW
GCH = 32
CCH = 32
SLOTC = 512
HD = DIM // 2


def _silu(v):
    return v * jax.nn.sigmoid(v)


def _mmT(a, b):
    return jax.lax.dot_general(a, b, (((1,), (1,)), ((), ())),
                               preferred_element_type=jnp.float32)


def _pack(v):
    u = lax.bitcast_convert_type(v, jnp.uint32)
    one = jnp.uint32(1)
    r = (u + jnp.uint32(0x7FFF) + ((u >> jnp.uint32(16)) & one)) >> jnp.uint32(16)
    w = r[:, :HD] | (r[:, HD:] << jnp.uint32(16))
    return lax.bitcast_convert_type(w, jnp.float32)


def _unpack(w):
    u = lax.bitcast_convert_type(w, jnp.uint32)
    lo = lax.bitcast_convert_type(u << jnp.uint32(16), jnp.float32)
    hi = lax.bitcast_convert_type(u & jnp.uint32(0xFFFF0000), jnp.float32)
    return jnp.concatenate([lo, hi], axis=1).astype(jnp.bfloat16)


def _route_body(x_ref, wg_ref, bias_ref,
                d0_ref, d1_ref, st_ref, sw_ref, be_ref, xw_ref):
    x = x_ref[...]
    xw_ref[...] = _pack(x)
    scores = jax.nn.sigmoid(_mmT(x, wg_ref[...]))
    biased = scores + bias_ref[...]
    lane = jax.lax.broadcasted_iota(jnp.int32, (T, E), 1)
    m0 = jnp.max(biased, axis=1, keepdims=True)
    i0 = jnp.min(jnp.where(biased == m0, lane, E), axis=1, keepdims=True)
    masked = jnp.where(lane == i0, -jnp.inf, biased)
    m1 = jnp.max(masked, axis=1, keepdims=True)
    i1 = jnp.min(jnp.where(masked == m1, lane, E), axis=1, keepdims=True)
    w0col = jnp.sum(jnp.where(lane == i0, scores, 0.0), axis=1,
                    keepdims=True)
    w1col = jnp.sum(jnp.where(lane == i1, scores, 0.0), axis=1,
                    keepdims=True)
    cnt = ((lane == i0).astype(jnp.float32)
           + (lane == i1).astype(jnp.float32))
    r2 = jax.lax.broadcasted_iota(jnp.int32, (T, T), 0)
    c2 = jax.lax.broadcasted_iota(jnp.int32, (T, T), 1)
    tri = (c2 <= r2).astype(jnp.bfloat16)
    inc = jax.lax.dot_general(tri, cnt.astype(jnp.bfloat16),
                              (((1,), (0,)), ((), ())),
                              preferred_element_type=jnp.float32)
    exc = inc - cnt
    counts = inc[T - 1:T, :]
    nb = jnp.floor((counts + (BLK - 1)) * (1.0 / BLK))
    r8 = jax.lax.broadcasted_iota(jnp.int32, (E, E), 0)
    c8 = jax.lax.broadcasted_iota(jnp.int32, (E, E), 1)
    su = (r8 < c8).astype(jnp.float32)
    offb = jax.lax.dot_general(nb, su, (((1,), (0,)), ((), ())),
                               preferred_element_type=jnp.float32)
    offs = offb * float(BLK)
    d0 = jnp.sum(jnp.where(lane == i0, exc + offs, 0.0), axis=1, keepdims=True)
    d1 = jnp.sum(jnp.where(lane == i1, exc + offs, 0.0), axis=1, keepdims=True)
    d0_ref[...] = d0.astype(jnp.int32)
    d1_ref[...] = d1.astype(jnp.int32)
    ti = jax.lax.broadcasted_iota(jnp.int32, (T, SLOTC), 0).astype(jnp.float32)
    qi = jax.lax.broadcasted_iota(jnp.int32, (T, SLOTC), 1).astype(jnp.float32)
    for cidx in range(P // SLOTC):
        q = qi + float(cidx * SLOTC)
        m0 = d0 == q
        m1 = d1 == q
        hit = jnp.sum(jnp.where(m0 | m1, 1.0, 0.0), axis=0, keepdims=True)
        stv = jnp.sum(jnp.where(m0 | m1, ti, 0.0), axis=0, keepdims=True)
        swv = (jnp.sum(jnp.where(m0, w0col, 0.0), axis=0, keepdims=True)
               + jnp.sum(jnp.where(m1, w1col, 0.0), axis=0, keepdims=True))
        pid = q[0:1, :]
        fb = pid - jnp.floor(pid * (1.0 / T)) * float(T)
        stv = stv + (1.0 - hit) * fb
        st_ref[pl.ds(cidx, 1), :] = stv.astype(jnp.int32)
        sw_ref[pl.ds(cidx, 1), :] = swv
    offb_col = jnp.sum(jnp.where(r8 == c8, jnp.broadcast_to(offb, (E, E)),
                                 0.0), axis=1, keepdims=True)
    jb = jax.lax.broadcasted_iota(jnp.int32, (E, NB), 1).astype(jnp.float32)
    be = jnp.sum((jb >= offb_col).astype(jnp.float32), axis=0,
                 keepdims=True) - 1.0
    be_ref[...] = be.astype(jnp.int32)


def _dispatch_body(stok_hbm, x_hbm, xs_hbm,
                   idx_v, rows0_v, rows1_v, sg0, sg1, sw0, sw1):
    c = lax.axis_index("c")
    s = lax.axis_index("s")
    slot0 = c * (P // NC) + s * SPW
    pltpu.sync_copy(stok_hbm.at[pl.ds(slot0, SPW)], idx_v)
    bufs = (rows0_v, rows1_v)
    gsems = (sg0, sg1)
    wsems = (sw0, sw1)
    nch = SPW // GCH
    gps = [None] * nch
    wrs = [None] * nch
    for k in range(nch):
        if k >= 2:
            wrs[k - 2].wait()
        gps[k] = pltpu.async_copy(x_hbm.at[idx_v.at[pl.ds(k * GCH, GCH)]],
                                  bufs[k % 2], gsems[k % 2])
        if k >= 1:
            gps[k - 1].wait()
            wrs[k - 1] = pltpu.async_copy(
                bufs[(k - 1) % 2],
                xs_hbm.at[pl.ds(slot0 + (k - 1) * GCH, GCH)],
                wsems[(k - 1) % 2])
    gps[nch - 1].wait()
    wrs[nch - 1] = pltpu.async_copy(
        bufs[(nch - 1) % 2], xs_hbm.at[pl.ds(slot0 + (nch - 1) * GCH, GCH)],
        wsems[(nch - 1) % 2])
    wrs[nch - 2].wait()
    wrs[nch - 1].wait()


def _sharedh_body(xw_ref, ws1_ref, ws3_ref, hs_ref):
    bf16 = jnp.bfloat16
    x = _unpack(xw_ref[...])
    h = (_silu(_mmT(x, ws1_ref[...].astype(bf16)))
         * _mmT(x, ws3_ref[...].astype(bf16)))
    hs_ref[...] = h.astype(bf16)


def _grouped_body(be_ref, xs_ref, w1_ref, w3_ref, w2_ref, wsl_ref, eo_ref):
    bf16 = jnp.bfloat16
    x = _unpack(xs_ref[...])
    h = (_silu(_mmT(x, w1_ref[0].astype(bf16)))
         * _mmT(x, w3_ref[0].astype(bf16)))
    eo = _mmT(h.astype(bf16), w2_ref[0].astype(bf16))
    eo_ref[...] = _pack(eo * wsl_ref[0])


def _combine_body(d0_hbm, d1_hbm, eos_hbm, y0_hbm, y1_hbm,
                  d0_v, d1_v, rows0_v, rows1_v, sg0, sg1, sw0, sw1):
    wid = lax.axis_index("c") * NS + lax.axis_index("s")
    base = wid * TPW
    pltpu.sync_copy(d0_hbm.at[pl.ds(base, TPW)], d0_v)
    pltpu.sync_copy(d1_hbm.at[pl.ds(base, TPW)], d1_v)
    nch = TPW // CCH
    units = ([(d0_v, y0_hbm, k) for k in range(nch)]
             + [(d1_v, y1_hbm, k) for k in range(nch)])
    bufs = (rows0_v, rows1_v)
    gsems = (sg0, sg1)
    wsems = (sw0, sw1)
    n = len(units)
    gps = [None] * n
    wrs = [None] * n
    for u in range(n):
        idx_v, out_hbm, k = units[u]
        if u >= 2:
            wrs[u - 2].wait()
        gps[u] = pltpu.async_copy(eos_hbm.at[idx_v.at[pl.ds(k * CCH, CCH)]],
                                  bufs[u % 2], gsems[u % 2])
        if u >= 1:
            pidx, pout, pk = units[u - 1]
            gps[u - 1].wait()
            wrs[u - 1] = pltpu.async_copy(
                bufs[(u - 1) % 2], pout.at[pl.ds(base + pk * CCH, CCH)],
                wsems[(u - 1) % 2])
    lidx, lout, lk = units[n - 1]
    gps[n - 1].wait()
    wrs[n - 1] = pltpu.async_copy(
        bufs[(n - 1) % 2], lout.at[pl.ds(base + lk * CCH, CCH)],
        wsems[(n - 1) % 2])
    wrs[n - 2].wait()
    wrs[n - 1].wait()


def _final_body(hs_ref, y0_ref, y1_ref, ws2_ref, o_ref):
    z = _mmT(hs_ref[...], ws2_ref[...].astype(jnp.bfloat16))
    y0 = _unpack(y0_ref[...]).astype(jnp.float32)
    y1 = _unpack(y1_ref[...]).astype(jnp.float32)
    o_ref[...] = z + y0 + y1


@jax.jit
def _run(x, Wg, expert_bias, W1, W2, W3, Ws1, Ws2, Ws3):
    shape = x.shape
    xf = x.reshape(-1, DIM)
    bias2 = expert_bias.reshape(1, E)
    f32 = jnp.float32
    i32 = jnp.int32
    bf16 = jnp.bfloat16

    d0, d1, st, sw, be, xw = pl.pallas_call(
        _route_body,
        out_shape=(
            jax.ShapeDtypeStruct((T, 1), i32),
            jax.ShapeDtypeStruct((T, 1), i32),
            jax.ShapeDtypeStruct((P // SLOTC, SLOTC), i32),
            jax.ShapeDtypeStruct((P // SLOTC, SLOTC), f32),
            jax.ShapeDtypeStruct((1, NB), i32),
            jax.ShapeDtypeStruct((T, HD), f32),
        ),
    )(xf, Wg, bias2)
    d0 = d0.reshape(T)
    d1 = d1.reshape(T)

    mesh = plsc.VectorSubcoreMesh(core_axis_name="c", subcore_axis_name="s",
                                  num_cores=NC, num_subcores=NS)
    xs = pl.kernel(
        _dispatch_body,
        out_type=jax.ShapeDtypeStruct((P, HD), f32),
        mesh=mesh,
        scratch_types=[
            pltpu.VMEM((SPW,), i32),
            pltpu.VMEM((GCH, HD), f32),
            pltpu.VMEM((GCH, HD), f32),
            pltpu.SemaphoreType.DMA,
            pltpu.SemaphoreType.DMA,
            pltpu.SemaphoreType.DMA,
            pltpu.SemaphoreType.DMA,
        ],
    )(st.reshape(P), xw)

    hs = pl.pallas_call(
        _sharedh_body,
        grid=(T // BT,),
        in_specs=[
            pl.BlockSpec((BT, HD), lambda i: (i, 0)),
            pl.BlockSpec((SH, DIM), lambda i: (0, 0)),
            pl.BlockSpec((SH, DIM), lambda i: (0, 0)),
        ],
        out_specs=pl.BlockSpec((BT, SH), lambda i: (i, 0)),
        out_shape=jax.ShapeDtypeStruct((T, SH), bf16),
    )(xw, Ws1, Ws3)

    eos = pl.pallas_call(
        _grouped_body,
        grid_spec=pltpu.PrefetchScalarGridSpec(
            num_scalar_prefetch=1,
            grid=(NB,),
            in_specs=[
                pl.BlockSpec((BLK, HD), lambda i, be: (i, 0)),
                pl.BlockSpec((1, INTER, DIM), lambda i, be: (be[i], 0, 0)),
                pl.BlockSpec((1, INTER, DIM), lambda i, be: (be[i], 0, 0)),
                pl.BlockSpec((1, DIM, INTER), lambda i, be: (be[i], 0, 0)),
                pl.BlockSpec((1, BLK, 1), lambda i, be: (i, 0, 0)),
            ],
            out_specs=pl.BlockSpec((BLK, HD), lambda i, be: (i, 0)),
        ),
        out_shape=jax.ShapeDtypeStruct((P, HD), f32),
    )(be.reshape(NB), xs, W1, W3, W2, sw.reshape(NB, BLK, 1))

    y0, y1 = pl.kernel(
        _combine_body,
        out_type=(
            jax.ShapeDtypeStruct((T, HD), f32),
            jax.ShapeDtypeStruct((T, HD), f32),
        ),
        mesh=mesh,
        scratch_types=[
            pltpu.VMEM((TPW,), i32),
            pltpu.VMEM((TPW,), i32),
            pltpu.VMEM((CCH, HD), f32),
            pltpu.VMEM((CCH, HD), f32),
            pltpu.SemaphoreType.DMA,
            pltpu.SemaphoreType.DMA,
            pltpu.SemaphoreType.DMA,
            pltpu.SemaphoreType.DMA,
        ],
    )(d0, d1, eos)

    out = pl.pallas_call(
        _final_body,
        grid=(T // BT,),
        in_specs=[
            pl.BlockSpec((BT, SH), lambda i: (i, 0)),
            pl.BlockSpec((BT, HD), lambda i: (i, 0)),
            pl.BlockSpec((BT, HD), lambda i: (i, 0)),
            pl.BlockSpec((DIM, SH), lambda i: (0, 0)),
        ],
        out_specs=pl.BlockSpec((BT, DIM), lambda i: (i, 0)),
        out_shape=jax.ShapeDtypeStruct((T, DIM), f32),
    )(hs, y0, y1, Ws2)

    return out.reshape(shape)


def kernel(x, Wg, expert_bias, W1, W2, W3, Ws1, Ws2, Ws3):
    return _run(x, Wg, expert_bias, W1, W2, W3, Ws1, Ws2, Ws3)


# speedup vs baseline: 1.7143x; 1.1436x over previous
import jax
import jax.numpy as jnp
from jax import lax
from jax.experimental import pallas as pl
from jax.experimental.pallas import tpu as pltpu
from jax.experimental.pallas import tpu_sc as plsc

DIM = 1024
E = 8
TOPK = 2
INTER = 512
SH = 1024
T = 2048
A = T * TOPK
BLK = 256
NB = A // BLK + E
P = NB * BLK
BT = 256
NC, NS = 2, 16
NW = NC * NS
TPW = T // NW
SPW = P // NW
GCH = 32
CCH = 32
SLOTC = 512
HD = DIM // 2


def _silu(v):
    return v * jax.nn.sigmoid(v)


def _mmT(a, b):
    return jax.lax.dot_general(a, b, (((1,), (1,)), ((), ())),
                               preferred_element_type=jnp.float32)


def _pack(v):
    u = lax.bitcast_convert_type(v, jnp.uint32)
    one = jnp.uint32(1)
    r = (u + jnp.uint32(0x7FFF) + ((u >> jnp.uint32(16)) & one)) >> jnp.uint32(16)
    w = r[:, :HD] | (r[:, HD:] << jnp.uint32(16))
    return lax.bitcast_convert_type(w, jnp.float32)


def _unpack(w):
    u = lax.bitcast_convert_type(w, jnp.uint32)
    lo = lax.bitcast_convert_type(u << jnp.uint32(16), jnp.float32)
    hi = lax.bitcast_convert_type(u & jnp.uint32(0xFFFF0000), jnp.float32)
    return jnp.concatenate([lo, hi], axis=1).astype(jnp.bfloat16)


def _route_body(x_ref, wg_ref, bias_ref,
                d0_ref, d1_ref, st_ref, w0_ref, w1_ref, be_ref, xw_ref):
    x = x_ref[...]
    xw_ref[...] = _pack(x)
    scores = jax.nn.sigmoid(_mmT(x, wg_ref[...]))
    biased = scores + bias_ref[...]
    lane = jax.lax.broadcasted_iota(jnp.int32, (T, E), 1)
    m0 = jnp.max(biased, axis=1, keepdims=True)
    i0 = jnp.min(jnp.where(biased == m0, lane, E), axis=1, keepdims=True)
    masked = jnp.where(lane == i0, -jnp.inf, biased)
    m1 = jnp.max(masked, axis=1, keepdims=True)
    i1 = jnp.min(jnp.where(masked == m1, lane, E), axis=1, keepdims=True)
    w0_ref[...] = jnp.sum(jnp.where(lane == i0, scores, 0.0), axis=1,
                          keepdims=True)
    w1_ref[...] = jnp.sum(jnp.where(lane == i1, scores, 0.0), axis=1,
                          keepdims=True)
    cnt = ((lane == i0).astype(jnp.float32)
           + (lane == i1).astype(jnp.float32))
    r2 = jax.lax.broadcasted_iota(jnp.int32, (T, T), 0)
    c2 = jax.lax.broadcasted_iota(jnp.int32, (T, T), 1)
    tri = (c2 <= r2).astype(jnp.bfloat16)
    inc = jax.lax.dot_general(tri, cnt.astype(jnp.bfloat16),
                              (((1,), (0,)), ((), ())),
                              preferred_element_type=jnp.float32)
    exc = inc - cnt
    counts = inc[T - 1:T, :]
    nb = jnp.floor((counts + (BLK - 1)) * (1.0 / BLK))
    r8 = jax.lax.broadcasted_iota(jnp.int32, (E, E), 0)
    c8 = jax.lax.broadcasted_iota(jnp.int32, (E, E), 1)
    su = (r8 < c8).astype(jnp.float32)
    offb = jax.lax.dot_general(nb, su, (((1,), (0,)), ((), ())),
                               preferred_element_type=jnp.float32)
    offs = offb * float(BLK)
    d0 = jnp.sum(jnp.where(lane == i0, exc + offs, 0.0), axis=1, keepdims=True)
    d1 = jnp.sum(jnp.where(lane == i1, exc + offs, 0.0), axis=1, keepdims=True)
    d0_ref[...] = d0.astype(jnp.int32)
    d1_ref[...] = d1.astype(jnp.int32)
    ti = jax.lax.broadcasted_iota(jnp.int32, (T, SLOTC), 0).astype(jnp.float32)
    qi = jax.lax.broadcasted_iota(jnp.int32, (T, SLOTC), 1).astype(jnp.float32)
    for cidx in range(P // SLOTC):
        q = qi + float(cidx * SLOTC)
        mm = (d0 == q) | (d1 == q)
        pid = q[0:1, :]
        fb = pid - jnp.floor(pid * (1.0 / T)) * float(T)
        stv = fb + jnp.sum(jnp.where(mm, ti - fb, 0.0), axis=0, keepdims=True)
        st_ref[pl.ds(cidx, 1), :] = stv.astype(jnp.int32)
    offb_col = jnp.sum(jnp.where(r8 == c8, jnp.broadcast_to(offb, (E, E)),
                                 0.0), axis=1, keepdims=True)
    jb = jax.lax.broadcasted_iota(jnp.int32, (E, NB), 1).astype(jnp.float32)
    be = jnp.sum((jb >= offb_col).astype(jnp.float32), axis=0,
                 keepdims=True) - 1.0
    be_ref[...] = be.astype(jnp.int32)


def _dispatch_body(stok_hbm, x_hbm, xs_hbm,
                   idx_v, rows0_v, rows1_v, sg0, sg1, sw0, sw1):
    c = lax.axis_index("c")
    s = lax.axis_index("s")
    slot0 = c * (P // NC) + s * SPW
    pltpu.sync_copy(stok_hbm.at[pl.ds(slot0, SPW)], idx_v)
    bufs = (rows0_v, rows1_v)
    gsems = (sg0, sg1)
    wsems = (sw0, sw1)
    nch = SPW // GCH
    gps = [None] * nch
    wrs = [None] * nch
    for k in range(nch):
        if k >= 2:
            wrs[k - 2].wait()
        gps[k] = pltpu.async_copy(x_hbm.at[idx_v.at[pl.ds(k * GCH, GCH)]],
                                  bufs[k % 2], gsems[k % 2])
        if k >= 1:
            gps[k - 1].wait()
            wrs[k - 1] = pltpu.async_copy(
                bufs[(k - 1) % 2],
                xs_hbm.at[pl.ds(slot0 + (k - 1) * GCH, GCH)],
                wsems[(k - 1) % 2])
    gps[nch - 1].wait()
    wrs[nch - 1] = pltpu.async_copy(
        bufs[(nch - 1) % 2], xs_hbm.at[pl.ds(slot0 + (nch - 1) * GCH, GCH)],
        wsems[(nch - 1) % 2])
    wrs[nch - 2].wait()
    wrs[nch - 1].wait()


def _sharedh_body(xw_ref, ws1_ref, ws3_ref, hs_ref):
    bf16 = jnp.bfloat16
    x = _unpack(xw_ref[...])
    h = (_silu(_mmT(x, ws1_ref[...].astype(bf16)))
         * _mmT(x, ws3_ref[...].astype(bf16)))
    hs_ref[...] = h.astype(bf16)


def _grouped_body(be_ref, xs_ref, w1_ref, w3_ref, w2_ref, eo_ref):
    bf16 = jnp.bfloat16
    x = _unpack(xs_ref[...])
    h = (_silu(_mmT(x, w1_ref[0].astype(bf16)))
         * _mmT(x, w3_ref[0].astype(bf16)))
    eo = _mmT(h.astype(bf16), w2_ref[0].astype(bf16))
    eo_ref[...] = _pack(eo)


def _combine_body(d0_hbm, d1_hbm, eos_hbm, y0_hbm, y1_hbm,
                  d0_v, d1_v, rows0_v, rows1_v, sg0, sg1, sw0, sw1):
    wid = lax.axis_index("c") * NS + lax.axis_index("s")
    base = wid * TPW
    pltpu.sync_copy(d0_hbm.at[pl.ds(base, TPW)], d0_v)
    pltpu.sync_copy(d1_hbm.at[pl.ds(base, TPW)], d1_v)
    nch = TPW // CCH
    units = ([(d0_v, y0_hbm, k) for k in range(nch)]
             + [(d1_v, y1_hbm, k) for k in range(nch)])
    bufs = (rows0_v, rows1_v)
    gsems = (sg0, sg1)
    wsems = (sw0, sw1)
    n = len(units)
    gps = [None] * n
    wrs = [None] * n
    for u in range(n):
        idx_v, out_hbm, k = units[u]
        if u >= 2:
            wrs[u - 2].wait()
        gps[u] = pltpu.async_copy(eos_hbm.at[idx_v.at[pl.ds(k * CCH, CCH)]],
                                  bufs[u % 2], gsems[u % 2])
        if u >= 1:
            pidx, pout, pk = units[u - 1]
            gps[u - 1].wait()
            wrs[u - 1] = pltpu.async_copy(
                bufs[(u - 1) % 2], pout.at[pl.ds(base + pk * CCH, CCH)],
                wsems[(u - 1) % 2])
    lidx, lout, lk = units[n - 1]
    gps[n - 1].wait()
    wrs[n - 1] = pltpu.async_copy(
        bufs[(n - 1) % 2], lout.at[pl.ds(base + lk * CCH, CCH)],
        wsems[(n - 1) % 2])
    wrs[n - 2].wait()
    wrs[n - 1].wait()


def _final_body(hs_ref, y0_ref, y1_ref, w0_ref, w1_ref, ws2_ref, o_ref):
    z = _mmT(hs_ref[...], ws2_ref[...].astype(jnp.bfloat16))
    y0 = _unpack(y0_ref[...]).astype(jnp.float32)
    y1 = _unpack(y1_ref[...]).astype(jnp.float32)
    o_ref[...] = z + w0_ref[...] * y0 + w1_ref[...] * y1


@jax.jit
def _run(x, Wg, expert_bias, W1, W2, W3, Ws1, Ws2, Ws3):
    shape = x.shape
    xf = x.reshape(-1, DIM)
    bias2 = expert_bias.reshape(1, E)
    f32 = jnp.float32
    i32 = jnp.int32
    bf16 = jnp.bfloat16

    d0, d1, st, w0, w1, be, xw = pl.pallas_call(
        _route_body,
        out_shape=(
            jax.ShapeDtypeStruct((T, 1), i32),
            jax.ShapeDtypeStruct((T, 1), i32),
            jax.ShapeDtypeStruct((P // SLOTC, SLOTC), i32),
            jax.ShapeDtypeStruct((T, 1), f32),
            jax.ShapeDtypeStruct((T, 1), f32),
            jax.ShapeDtypeStruct((1, NB), i32),
            jax.ShapeDtypeStruct((T, HD), f32),
        ),
    )(xf, Wg, bias2)
    d0 = d0.reshape(T)
    d1 = d1.reshape(T)

    mesh = plsc.VectorSubcoreMesh(core_axis_name="c", subcore_axis_name="s",
                                  num_cores=NC, num_subcores=NS)
    xs = pl.kernel(
        _dispatch_body,
        out_type=jax.ShapeDtypeStruct((P, HD), f32),
        mesh=mesh,
        scratch_types=[
            pltpu.VMEM((SPW,), i32),
            pltpu.VMEM((GCH, HD), f32),
            pltpu.VMEM((GCH, HD), f32),
            pltpu.SemaphoreType.DMA,
            pltpu.SemaphoreType.DMA,
            pltpu.SemaphoreType.DMA,
            pltpu.SemaphoreType.DMA,
        ],
    )(st.reshape(P), xw)

    hs = pl.pallas_call(
        _sharedh_body,
        grid=(T // BT,),
        in_specs=[
            pl.BlockSpec((BT, HD), lambda i: (i, 0)),
            pl.BlockSpec((SH, DIM), lambda i: (0, 0)),
            pl.BlockSpec((SH, DIM), lambda i: (0, 0)),
        ],
        out_specs=pl.BlockSpec((BT, SH), lambda i: (i, 0)),
        out_shape=jax.ShapeDtypeStruct((T, SH), bf16),
    )(xw, Ws1, Ws3)

    eos = pl.pallas_call(
        _grouped_body,
        grid_spec=pltpu.PrefetchScalarGridSpec(
            num_scalar_prefetch=1,
            grid=(NB,),
            in_specs=[
                pl.BlockSpec((BLK, HD), lambda i, be: (i, 0)),
                pl.BlockSpec((1, INTER, DIM), lambda i, be: (be[i], 0, 0)),
                pl.BlockSpec((1, INTER, DIM), lambda i, be: (be[i], 0, 0)),
                pl.BlockSpec((1, DIM, INTER), lambda i, be: (be[i], 0, 0)),
            ],
            out_specs=pl.BlockSpec((BLK, HD), lambda i, be: (i, 0)),
        ),
        out_shape=jax.ShapeDtypeStruct((P, HD), f32),
    )(be.reshape(NB), xs, W1, W3, W2)

    y0, y1 = pl.kernel(
        _combine_body,
        out_type=(
            jax.ShapeDtypeStruct((T, HD), f32),
            jax.ShapeDtypeStruct((T, HD), f32),
        ),
        mesh=mesh,
        scratch_types=[
            pltpu.VMEM((TPW,), i32),
            pltpu.VMEM((TPW,), i32),
            pltpu.VMEM((CCH, HD), f32),
            pltpu.VMEM((CCH, HD), f32),
            pltpu.SemaphoreType.DMA,
            pltpu.SemaphoreType.DMA,
            pltpu.SemaphoreType.DMA,
            pltpu.SemaphoreType.DMA,
        ],
    )(d0, d1, eos)

    out = pl.pallas_call(
        _final_body,
        grid=(T // BT,),
        in_specs=[
            pl.BlockSpec((BT, SH), lambda i: (i, 0)),
            pl.BlockSpec((BT, HD), lambda i: (i, 0)),
            pl.BlockSpec((BT, HD), lambda i: (i, 0)),
            pl.BlockSpec((BT, 1), lambda i: (i, 0)),
            pl.BlockSpec((BT, 1), lambda i: (i, 0)),
            pl.BlockSpec((DIM, SH), lambda i: (0, 0)),
        ],
        out_specs=pl.BlockSpec((BT, DIM), lambda i: (i, 0)),
        out_shape=jax.ShapeDtypeStruct((T, DIM), f32),
    )(hs, y0, y1, w0, w1, Ws2)

    return out.reshape(shape)


def kernel(x, Wg, expert_bias, W1, W2, W3, Ws1, Ws2, Ws3):
    return _run(x, Wg, expert_bias, W1, W2, W3, Ws1, Ws2, Ws3)
